# baseline (speedup 1.0000x reference)
import functools

import jax
import jax.numpy as jnp
from jax import lax
from jax.experimental import pallas as pl
from jax.experimental.pallas import tpu as pltpu
from jax.experimental.pallas import tpu_sc as plsc

NC = 2
NS = 16
L = 16
NW = NC * NS


def _emb_body(B, S, D, PW, ids_hbm, word_hbm, pos_hbm, type_hbm, gamma_hbm,
              beta_hbm, out_hbm, idx_v, rows_v, pos_v, type_v, gamma_v,
              beta_v, sem):
    J = D // L
    wid = lax.axis_index("s") * NC + lax.axis_index("c")
    pos0 = wid * PW

    pltpu.sync_copy(pos_hbm.at[pl.ds(pos0, PW)], pos_v)
    pltpu.sync_copy(type_hbm.at[0], type_v)
    pltpu.sync_copy(gamma_hbm, gamma_v)
    pltpu.sync_copy(beta_hbm, beta_v)

    for j in range(J):
        sl = pl.ds(j * L, L)
        tv = type_v[sl]

        def pbody(p, tv):
            pos_v[p, sl] = pos_v[p, sl] + tv
            return tv

        lax.fori_loop(0, PW, pbody, tv)

    for b in range(B):
        base = b * S + pos0
        pltpu.sync_copy(ids_hbm.at[pl.ds(base, PW)], idx_v)
        pltpu.async_copy(word_hbm.at[idx_v], rows_v, sem).wait()

        def tbody(t, carry):
            acc = jnp.zeros((L,), jnp.float32)
            acc2 = jnp.zeros((L,), jnp.float32)
            for j in range(J):
                sl = pl.ds(j * L, L)
                v = rows_v[t, sl] + pos_v[t, sl]
                rows_v[t, sl] = v
                acc = acc + v
                acc2 = acc2 + v * v
            s1 = jnp.broadcast_to(jnp.sum(acc), (L,))
            s2 = jnp.broadcast_to(jnp.sum(acc2), (L,))
            mean = s1 * (1.0 / D)
            var = s2 * (1.0 / D) - mean * mean
            x = var + 1e-12
            xi = lax.bitcast_convert_type(x, jnp.int32)
            yi = jnp.int32(0x5F3759DF) - lax.shift_right_logical(xi, 1)
            y = lax.bitcast_convert_type(yi, jnp.float32)
            hx = x * 0.5
            for _ in range(3):
                y = y * (1.5 - hx * y * y)
            for j in range(J):
                sl = pl.ds(j * L, L)
                v = rows_v[t, sl]
                rows_v[t, sl] = (v - mean) * y * gamma_v[sl] + beta_v[sl]
            return carry

        lax.fori_loop(0, PW, tbody, 0)
        pltpu.sync_copy(rows_v, out_hbm.at[pl.ds(base, PW)])


def kernel(input_ids, word_emb, pos_emb, type_emb, ln_gamma, ln_beta):
    B, S = input_ids.shape
    V, D = word_emb.shape
    assert S % NW == 0 and D % L == 0
    PW = S // NW

    mesh = plsc.VectorSubcoreMesh(
        core_axis_name="c", subcore_axis_name="s", num_cores=NC,
        num_subcores=NS)
    fn = pl.kernel(
        functools.partial(_emb_body, B, S, D, PW),
        out_type=jax.ShapeDtypeStruct((B * S, D), jnp.float32),
        mesh=mesh,
        compiler_params=pltpu.CompilerParams(needs_layout_passes=False),
        scratch_types=[
            pltpu.VMEM((PW,), jnp.int32),
            pltpu.VMEM((PW, D), jnp.float32),
            pltpu.VMEM((PW, D), jnp.float32),
            pltpu.VMEM((D,), jnp.float32),
            pltpu.VMEM((D,), jnp.float32),
            pltpu.VMEM((D,), jnp.float32),
            pltpu.SemaphoreType.DMA,
        ],
    )
    out = fn(input_ids.reshape(B * S), word_emb, pos_emb, type_emb,
             ln_gamma, ln_beta)
    return out.reshape(B, S, D)
